# Initial kernel scaffold; baseline (speedup 1.0000x reference)
import jax
import jax.numpy as jnp
from jax.experimental import pallas as pl


def kernel(sample, relation_embedding, entity_embedding, neg):
    raise NotImplementedError("write your pallas kernel here")


# speedup vs baseline: 1.1377x; 1.1377x over previous
import jax
import jax.numpy as jnp
from jax import lax
from jax.experimental import pallas as pl
from jax.experimental.pallas import tpu as pltpu
from jax.experimental.pallas import tpu_sc as plsc

_GAMMA = 10.0
_L = 16


def _score_sc(hidx, ridx, tidx, rel2d, ent2d):
    B = hidx.shape[0]
    NC, NS = 2, 16
    NW = NC * NS
    BPW = B // NW
    CH = 128
    NCH = BPW // CH
    NG = BPW // _L

    mesh = plsc.VectorSubcoreMesh(
        core_axis_name="c", subcore_axis_name="s", num_cores=NC, num_subcores=NS
    )

    def body(hidx_hbm, ridx_hbm, tidx_hbm, rel_hbm, ent_hbm, out_hbm,
             hidx_v, ridx_v, tidx_v, h_v, r_v, t_v, out_v, sem):
        wid = lax.axis_index("s") * NC + lax.axis_index("c")
        base = wid * BPW

        pltpu.sync_copy(hidx_hbm.at[pl.ds(base, BPW)], hidx_v)
        pltpu.sync_copy(ridx_hbm.at[pl.ds(base, BPW)], ridx_v)
        pltpu.sync_copy(tidx_hbm.at[pl.ds(base, BPW)], tidx_v)

        copies = []
        for j in range(NCH):
            sl = pl.ds(j * CH, CH)
            copies.append(pltpu.async_copy(ent_hbm.at[hidx_v.at[sl]], h_v.at[sl], sem))
            copies.append(pltpu.async_copy(ent_hbm.at[tidx_v.at[sl]], t_v.at[sl], sem))
            copies.append(pltpu.async_copy(rel_hbm.at[ridx_v.at[sl]], r_v.at[sl], sem))
        for c in copies:
            c.wait()

        lane = lax.iota(jnp.int32, _L)

        def group_body(g, _):
            i0 = g * _L
            acc = jnp.zeros((_L,), jnp.float32)
            for l in range(_L):
                i = i0 + l
                d0 = jnp.abs(h_v[i, 0:16] + r_v[i, 0:16] - t_v[i, 0:16])
                d1 = jnp.abs(h_v[i, 16:32] + r_v[i, 16:32] - t_v[i, 16:32])
                d2 = jnp.abs(h_v[i, 32:48] - t_v[i, 32:48])
                d3 = jnp.abs(h_v[i, 48:64] - t_v[i, 48:64])
                s = jnp.sum((d0 + d1) + (d2 + d3))
                acc = jnp.where(lane == l, s, acc)
            out_v[pl.ds(i0, _L)] = _GAMMA - acc
            return 0

        lax.fori_loop(0, NG, group_body, 0)

        pltpu.sync_copy(out_v, out_hbm.at[pl.ds(base, BPW)])

    return pl.kernel(
        body,
        out_type=jax.ShapeDtypeStruct((B,), jnp.float32),
        mesh=mesh,
        compiler_params=pltpu.CompilerParams(
            needs_layout_passes=False, use_tc_tiling_on_sc=False
        ),
        scratch_types=[
            pltpu.VMEM((BPW,), jnp.int32),
            pltpu.VMEM((BPW,), jnp.int32),
            pltpu.VMEM((BPW,), jnp.int32),
            pltpu.VMEM((BPW, 64), jnp.float32),
            pltpu.VMEM((BPW, 32), jnp.float32),
            pltpu.VMEM((BPW, 64), jnp.float32),
            pltpu.VMEM((BPW,), jnp.float32),
            pltpu.SemaphoreType.DMA,
        ],
    )(hidx, ridx, tidx, rel2d, ent2d)


def kernel(sample, relation_embedding, entity_embedding, neg):
    del neg
    ent2d = entity_embedding.reshape(entity_embedding.shape[0], -1)
    rel2d = relation_embedding.reshape(relation_embedding.shape[0], -1)
    idx = sample.astype(jnp.int32)
    return _score_sc(idx[:, 0], idx[:, 1], idx[:, 2], rel2d, ent2d)
